# baseline (speedup 1.0000x reference)
import functools

import jax
import jax.numpy as jnp
from jax import lax
from jax.experimental import pallas as pl
from jax.experimental.pallas import tpu as pltpu
from jax.experimental.pallas import tpu_sc as plsc

_MESH = plsc.VectorSubcoreMesh(core_axis_name="c", subcore_axis_name="s")


@functools.partial(
    pl.kernel,
    mesh=_MESH,
    out_type=[
        jax.ShapeDtypeStruct((1, 3), jnp.float32),
        jax.ShapeDtypeStruct((1, 3), jnp.float32),
    ],
    scratch_types=[
        pltpu.VMEM((16,), jnp.int32),
        pltpu.VMEM((1, 3), jnp.float32),
        pltpu.VMEM((1, 3), jnp.float32),
        pltpu.SemaphoreType.DMA,
    ],
)
def _sc_lookup(idx_hbm, rot_hbm, trans_hbm, theta_hbm, rho_hbm,
               idx_v, theta_v, rho_v, sem):
    first = jnp.logical_and(lax.axis_index("c") == 0, lax.axis_index("s") == 0)

    @pl.when(first)
    def _():
        pltpu.sync_copy(idx_hbm, idx_v.at[pl.ds(0, 1)])
        i = idx_v[...][0]
        a = pltpu.make_async_copy(rot_hbm.at[pl.ds(i, 1)], theta_v, sem)
        b = pltpu.make_async_copy(trans_hbm.at[pl.ds(i, 1)], rho_v, sem)
        a.start()
        b.start()
        a.wait()
        b.wait()
        pltpu.sync_copy(theta_v, theta_hbm)
        pltpu.sync_copy(rho_v, rho_hbm)


def kernel(view_ids, rot_weight, trans_weight):
    idx = view_ids[:1].astype(jnp.int32)
    theta, rho = _sc_lookup(idx, rot_weight, trans_weight)
    return (theta, rho)


# speedup vs baseline: 1.0341x; 1.0341x over previous
import functools

import jax
import jax.numpy as jnp
from jax import lax
from jax.experimental import pallas as pl
from jax.experimental.pallas import tpu as pltpu
from jax.experimental.pallas import tpu_sc as plsc

_MESH = plsc.ScalarSubcoreMesh(axis_name="c", num_cores=2)


@functools.partial(
    pl.kernel,
    mesh=_MESH,
    out_type=[
        jax.ShapeDtypeStruct((1, 3), jnp.float32),
        jax.ShapeDtypeStruct((1, 3), jnp.float32),
    ],
    scratch_types=[
        pltpu.SMEM((1,), jnp.int32),
        pltpu.SMEM((1, 3), jnp.float32),
        pltpu.SMEM((1, 3), jnp.float32),
        pltpu.SemaphoreType.DMA,
    ],
)
def _sc_lookup(idx_hbm, rot_hbm, trans_hbm, theta_hbm, rho_hbm,
               idx_s, theta_s, rho_s, sem):
    @pl.when(lax.axis_index("c") == 0)
    def _():
        pltpu.sync_copy(idx_hbm, idx_s)
        i = idx_s[0]
        a = pltpu.make_async_copy(rot_hbm.at[pl.ds(i, 1)], theta_s, sem)
        b = pltpu.make_async_copy(trans_hbm.at[pl.ds(i, 1)], rho_s, sem)
        a.start()
        b.start()
        a.wait()
        b.wait()
        pltpu.sync_copy(theta_s, theta_hbm)
        pltpu.sync_copy(rho_s, rho_hbm)


def kernel(view_ids, rot_weight, trans_weight):
    idx = view_ids[:1].astype(jnp.int32)
    theta, rho = _sc_lookup(idx, rot_weight, trans_weight)
    return (theta, rho)


# speedup vs baseline: 3.0332x; 2.9332x over previous
import jax
import jax.numpy as jnp
from jax.experimental import pallas as pl
from jax.experimental.pallas import tpu as pltpu


def _vec3(a, b, c):
    lane = jax.lax.broadcasted_iota(jnp.int32, (1, 3), 1)
    return jnp.where(lane == 0, a, jnp.where(lane == 1, b, c))


def _lookup_kernel(idx_ref, rot_ref, trans_ref, theta_ref, rho_ref):
    i = idx_ref[0]
    theta_ref[...] = _vec3(rot_ref[i, 0], rot_ref[i, 1], rot_ref[i, 2])
    rho_ref[...] = _vec3(trans_ref[i, 0], trans_ref[i, 1], trans_ref[i, 2])


def kernel(view_ids, rot_weight, trans_weight):
    idx = view_ids[:1].astype(jnp.int32)
    theta, rho = pl.pallas_call(
        _lookup_kernel,
        in_specs=[
            pl.BlockSpec(memory_space=pltpu.SMEM),
            pl.BlockSpec(memory_space=pltpu.SMEM),
            pl.BlockSpec(memory_space=pltpu.SMEM),
        ],
        out_specs=[
            pl.BlockSpec(memory_space=pltpu.VMEM),
            pl.BlockSpec(memory_space=pltpu.VMEM),
        ],
        out_shape=[
            jax.ShapeDtypeStruct((1, 3), jnp.float32),
            jax.ShapeDtypeStruct((1, 3), jnp.float32),
        ],
    )(idx, rot_weight, trans_weight)
    return (theta, rho)


# speedup vs baseline: 4.5432x; 1.4978x over previous
import jax
import jax.numpy as jnp
from jax.experimental import pallas as pl
from jax.experimental.pallas import tpu as pltpu


def _lookup_kernel(idx_ref, rot_ref, trans_ref, theta_ref, rho_ref,
                   theta_v, rho_v, sem1, sem2):
    i = idx_ref[0]
    a = pltpu.make_async_copy(rot_ref.at[pl.ds(i, 1)], theta_v, sem1)
    b = pltpu.make_async_copy(trans_ref.at[pl.ds(i, 1)], rho_v, sem2)
    a.start()
    b.start()
    a.wait()
    b.wait()
    theta_ref[...] = theta_v[...]
    rho_ref[...] = rho_v[...]


def kernel(view_ids, rot_weight, trans_weight):
    idx = view_ids[:1].astype(jnp.int32)
    theta, rho = pl.pallas_call(
        _lookup_kernel,
        in_specs=[
            pl.BlockSpec(memory_space=pltpu.SMEM),
            pl.BlockSpec(memory_space=pl.ANY),
            pl.BlockSpec(memory_space=pl.ANY),
        ],
        out_specs=[
            pl.BlockSpec(memory_space=pltpu.VMEM),
            pl.BlockSpec(memory_space=pltpu.VMEM),
        ],
        out_shape=[
            jax.ShapeDtypeStruct((1, 3), jnp.float32),
            jax.ShapeDtypeStruct((1, 3), jnp.float32),
        ],
        scratch_shapes=[
            pltpu.VMEM((1, 3), jnp.float32),
            pltpu.VMEM((1, 3), jnp.float32),
            pltpu.SemaphoreType.DMA,
            pltpu.SemaphoreType.DMA,
        ],
    )(idx, rot_weight, trans_weight)
    return (theta, rho)
